# baseline (speedup 1.0000x reference)
import jax
import jax.numpy as jnp
from jax import lax
from jax.experimental import pallas as pl
from jax.experimental.pallas import tpu as pltpu
from jax.experimental.pallas import tpu_sc as plsc

B = 4096
SEQ = 200
D = 64
V = 1000000
NW = 32
CHUNK = 128
DEPTH = 4
K = DEPTH - 1
S_SPLIT = (96, 104)


def _make_body(per_w, nchunk):
    def _emb_body(ids_hbm, table_hbm, out_hbm, idx_v, rows_v, gsem, osem):
        wid = lax.axis_index("s") * 2 + lax.axis_index("c")
        base = wid * per_w
        pltpu.sync_copy(ids_hbm.at[pl.ds(base, per_w)], idx_v)

        def gather(i, b):
            def grp(g, _):
                jv = idx_v[pl.ds(i * CHUNK + g * 16, 16)]
                jhi = lax.shift_right_logical(jv, 3)
                jlo = lax.rem(jv, 8)
                for t16 in range(16):
                    pltpu.async_copy(
                        table_hbm.at[jhi[t16], jlo[t16]],
                        rows_v.at[b, g * 16 + t16],
                        gsem.at[b],
                    )
                return 0

            lax.fori_loop(0, CHUNK // 16, grp, 0, unroll=False)

        def wait_gather(b):
            pltpu.make_async_copy(
                out_hbm.at[pl.ds(0, CHUNK)], rows_v.at[b], gsem.at[b]
            ).wait()

        def put(i, b):
            pltpu.async_copy(
                rows_v.at[b],
                out_hbm.at[pl.ds(base + i * CHUNK, CHUNK)],
                osem.at[b],
            )

        def wait_put(b):
            pltpu.make_async_copy(
                rows_v.at[b],
                out_hbm.at[pl.ds(base, CHUNK)],
                osem.at[b],
            ).wait()

        for j in range(K):
            gather(j, j)

        def body(g, _):
            for db in range(DEPTH):
                i = g * DEPTH + db
                b = db
                bn = (db + K) % DEPTH
                @pl.when(i + K < nchunk)
                def _():
                    @pl.when(i >= 1)
                    def _():
                        wait_put(bn)

                    gather(i + K, bn)

                wait_gather(b)
                put(i, b)
            return 0

        lax.fori_loop(0, nchunk // DEPTH, body, 0, unroll=False)

        for c in range(nchunk - DEPTH, nchunk):
            wait_put(c % DEPTH)

    return _emb_body


@jax.jit
def kernel(token_ids, embed_weight):
    table_view = embed_weight.reshape(V // 8, 8, D)
    mesh = plsc.VectorSubcoreMesh(core_axis_name="c", subcore_axis_name="s")
    halves = []
    s0 = 0
    for s_len in S_SPLIT:
        n_h = B * s_len
        per_w = n_h // NW
        nchunk = per_w // CHUNK
        ids_h = token_ids[:, s0 : s0 + s_len].reshape(-1)
        out_h = pl.kernel(
            _make_body(per_w, nchunk),
            out_type=jax.ShapeDtypeStruct((n_h, D), jnp.float32),
            mesh=mesh,
            scratch_types=[
                pltpu.VMEM((per_w,), jnp.int32),
                pltpu.VMEM((DEPTH, CHUNK, D), jnp.float32),
                pltpu.SemaphoreType.DMA((DEPTH,)),
                pltpu.SemaphoreType.DMA((DEPTH,)),
            ],
        )(ids_h, table_view)
        halves.append(out_h.reshape(B, s_len, D))
        s0 += s_len
    return jnp.concatenate(halves, axis=1)


# speedup vs baseline: 1.1934x; 1.1934x over previous
import jax
import jax.numpy as jnp
from jax import lax
from jax.experimental import pallas as pl
from jax.experimental.pallas import tpu as pltpu
from jax.experimental.pallas import tpu_sc as plsc

B = 4096
SEQ = 200
D = 64
V = 1000000
N = B * SEQ
NW = 32
PER_W = N // NW
CHUNK = 128
NCHUNK = PER_W // CHUNK
DEPTH = 5
K = DEPTH - 1


def _emb_body(ids_hbm, table_hbm, out_hbm, idx_v, rows_v, gsem, osem):
    wid = lax.axis_index("s") * 2 + lax.axis_index("c")
    base = wid * PER_W
    pltpu.sync_copy(ids_hbm.at[pl.ds(base, PER_W)], idx_v)

    def gather(i, b):
        def grp(g, _):
            jv = idx_v[pl.ds(i * CHUNK + g * 16, 16)]
            jhi = lax.shift_right_logical(jv, 3)
            jlo = lax.rem(jv, 8)
            for t16 in range(16):
                pltpu.async_copy(
                    table_hbm.at[jhi[t16], jlo[t16]],
                    rows_v.at[b, g * 16 + t16],
                    gsem.at[b],
                )
            return 0

        lax.fori_loop(0, CHUNK // 16, grp, 0, unroll=False)

    def wait_gather(b):
        pltpu.make_async_copy(
            out_hbm.at[pl.ds(0, CHUNK)], rows_v.at[b], gsem.at[b]
        ).wait()

    def put(i, b):
        pltpu.async_copy(
            rows_v.at[b],
            out_hbm.at[pl.ds(base + i * CHUNK, CHUNK)],
            osem.at[b],
        )

    def wait_put(b):
        pltpu.make_async_copy(
            rows_v.at[b],
            out_hbm.at[pl.ds(base, CHUNK)],
            osem.at[b],
        ).wait()

    for j in range(K):
        gather(j, j)

    def body(g, _):
        for db in range(DEPTH):
            i = g * DEPTH + db
            b = db
            bn = (db + K) % DEPTH
            @pl.when(i + K < NCHUNK)
            def _():
                @pl.when(i >= 1)
                def _():
                    wait_put(bn)

                gather(i + K, bn)

            wait_gather(b)
            put(i, b)
        return 0

    lax.fori_loop(0, NCHUNK // DEPTH, body, 0, unroll=False)

    for c in range(NCHUNK - DEPTH, NCHUNK):
        wait_put(c % DEPTH)


@jax.jit
def kernel(token_ids, embed_weight):
    ids_flat = token_ids.reshape(-1)
    mesh = plsc.VectorSubcoreMesh(core_axis_name="c", subcore_axis_name="s")
    out = pl.kernel(
        _emb_body,
        out_type=jax.ShapeDtypeStruct((N, D), jnp.float32),
        mesh=mesh,
        scratch_types=[
            pltpu.VMEM((PER_W,), jnp.int32),
            pltpu.VMEM((DEPTH, CHUNK, D), jnp.float32),
            pltpu.SemaphoreType.DMA((DEPTH,)),
            pltpu.SemaphoreType.DMA((DEPTH,)),
        ],
    )(ids_flat, embed_weight.reshape(V // 8, 8, D))
    return out.reshape(B, SEQ, D)
